# baseline (speedup 1.0000x reference)
import jax
import jax.numpy as jnp
from jax import lax
from jax.experimental import pallas as pl
from jax.experimental.pallas import tpu as pltpu
from jax.experimental.pallas import tpu_sc as plsc

N = 10000
E = 320000
D = 128
EDGE_DIM = 16
M_DIM = 64
N_LAYERS = 3

NC = 2
NS = 16
NW = NC * NS

CH = 128
H = 2
NCHUNK = 40
PT = CH * NCHUNK
E_SL = PT * NW
E_PAD = E_SL * H
N_PAD = 10240
RPT = N_PAD // NS

TWT = 128
SW = 128

EB = 2048
NB = 512

_MESH = plsc.VectorSubcoreMesh(
    core_axis_name="c", subcore_axis_name="s", num_cores=NC, num_subcores=NS)


def _sc_gather_body(a_hbm, b_hbm, dst_hbm, src_hbm, g_hbm,
                    idx_d, idx_s, buf_a0, buf_b0, buf_a1, buf_b1,
                    sem_a0, sem_b0, sem_a1, sem_b1):
    c = lax.axis_index("c")
    s = lax.axis_index("s")
    wid = s * NC + c
    ebase = wid * PT

    pltpu.sync_copy(dst_hbm.at[wid], idx_d)
    pltpu.sync_copy(src_hbm.at[wid], idx_s)

    slots = ((buf_a0, buf_b0, sem_a0, sem_b0),
             (buf_a1, buf_b1, sem_a1, sem_b1))

    def issue(slot, k):
        ba, bb, sa, sb = slots[slot]
        pltpu.async_copy(a_hbm.at[idx_d.at[k]], ba, sa)
        pltpu.async_copy(b_hbm.at[idx_s.at[k]], bb, sb)

    def process(slot, k):
        ba, bb, sa, sb = slots[slot]
        pltpu.make_async_copy(a_hbm.at[idx_d.at[k]], ba, sa).wait()
        pltpu.make_async_copy(b_hbm.at[idx_s.at[k]], bb, sb).wait()

        @pl.loop(0, CH, unroll=4)
        def addrow(r):
            for j in range(5):
                sl = pl.ds(j * 16, 16)
                ba[r, sl] = ba[r, sl] + bb[r, sl]

        pltpu.sync_copy(ba, g_hbm.at[pl.ds(ebase + k * CH, CH)])

    issue(0, 0)

    @pl.loop(0, (NCHUNK - 2) // 2)
    def it(j):
        k0 = 2 * j
        issue(1, k0 + 1)
        process(0, k0)
        issue(0, k0 + 2)
        process(1, k0 + 1)

    issue(1, NCHUNK - 1)
    process(0, NCHUNK - 2)
    process(1, NCHUNK - 1)


_sc_gather = pl.kernel(
    _sc_gather_body,
    out_type=jax.ShapeDtypeStruct((E_SL, TWT), jnp.float32),
    mesh=_MESH,
    scratch_types=[
        pltpu.VMEM((NCHUNK, CH), jnp.int32),
        pltpu.VMEM((NCHUNK, CH), jnp.int32),
        pltpu.VMEM((CH, TWT), jnp.float32),
        pltpu.VMEM((CH, TWT), jnp.float32),
        pltpu.VMEM((CH, TWT), jnp.float32),
        pltpu.VMEM((CH, TWT), jnp.float32),
        pltpu.SemaphoreType.DMA,
        pltpu.SemaphoreType.DMA,
        pltpu.SemaphoreType.DMA,
        pltpu.SemaphoreType.DMA,
    ],
)


def _sc_scatter_body(s_hbm, dst_hbm, zero_hbm, p_hbm, idx, buf0, buf1,
                     sem0, sem1, acc):
    c = lax.axis_index("c")
    s = lax.axis_index("s")
    wid = s * NC + c
    ebase = wid * PT

    pltpu.sync_copy(zero_hbm.at[pl.ds(s * RPT, RPT)], acc.at[pl.ds(s * RPT, RPT)])
    pltpu.sync_copy(dst_hbm.at[wid], idx)
    plsc.subcore_barrier()

    slots = ((buf0, sem0), (buf1, sem1))

    def issue(slot, k):
        b, sem = slots[slot]
        pltpu.async_copy(s_hbm.at[pl.ds(ebase + k * CH, CH)], b, sem)

    def process(slot, k):
        b, sem = slots[slot]
        pltpu.make_async_copy(s_hbm.at[pl.ds(ebase + k * CH, CH)], b, sem).wait()
        pltpu.sync_copy(b, acc.at[idx.at[k]], add=True)

    issue(0, 0)

    @pl.loop(0, (NCHUNK - 2) // 2)
    def it(j):
        k0 = 2 * j
        issue(1, k0 + 1)
        process(0, k0)
        issue(0, k0 + 2)
        process(1, k0 + 1)

    issue(1, NCHUNK - 1)
    process(0, NCHUNK - 2)
    process(1, NCHUNK - 1)

    plsc.subcore_barrier()
    pltpu.sync_copy(acc.at[pl.ds(s * RPT, RPT)], p_hbm.at[c, pl.ds(s * RPT, RPT)])


_sc_scatter = pl.kernel(
    _sc_scatter_body,
    out_type=jax.ShapeDtypeStruct((NC, N_PAD, SW), jnp.float32),
    mesh=_MESH,
    scratch_types=[
        pltpu.VMEM((NCHUNK, CH), jnp.int32),
        pltpu.VMEM((CH, SW), jnp.float32),
        pltpu.VMEM((CH, SW), jnp.float32),
        pltpu.SemaphoreType.DMA,
        pltpu.SemaphoreType.DMA,
        pltpu.MemorySpace.VMEM_SHARED((N_PAD, SW), jnp.float32),
    ],
)


def _tc_edge_body(g_ref, ea_ref, wr_ref, wea_ref, be1_ref, we2_ref, be2_ref,
                  wc1_ref, bc1_ref, wc2_ref, bc2_ref, s_ref):
    g = g_ref[...]
    rel = g[:, 0:3]
    q = g[:, 3:3 + M_DIM]
    rd = jnp.sum(rel * rel, axis=1, keepdims=True)
    pre = (q + rd * wr_ref[...]
           + jnp.dot(ea_ref[...], wea_ref[...], preferred_element_type=jnp.float32)
           + be1_ref[...])
    m = jax.nn.silu(pre)
    m = jax.nn.silu(jnp.dot(m, we2_ref[...], preferred_element_type=jnp.float32)
                    + be2_ref[...])
    cwh = jax.nn.silu(jnp.dot(m, wc1_ref[...], preferred_element_type=jnp.float32)
                      + bc1_ref[...])
    cw = jnp.dot(cwh, wc2_ref[...], preferred_element_type=jnp.float32) + bc2_ref[...]
    ones = jnp.ones((m.shape[0], 1), jnp.float32)
    pad = jnp.zeros((m.shape[0], SW - M_DIM - 4), jnp.float32)
    s_ref[...] = jnp.concatenate([m, cw * rel, ones, pad], axis=1)


def _tc_edge(gbuf, ea, wr, wea, be1, we2, be2, wc1, bc1, wc2, bc2):
    grid = (E_SL // EB,)
    full = lambda shape: pl.BlockSpec(shape, lambda i: (0, 0))
    return pl.pallas_call(
        _tc_edge_body,
        grid=grid,
        in_specs=[
            pl.BlockSpec((EB, TWT), lambda i: (i, 0)),
            pl.BlockSpec((EB, EDGE_DIM), lambda i: (i, 0)),
            full((1, M_DIM)), full((EDGE_DIM, M_DIM)), full((1, M_DIM)),
            full((M_DIM, M_DIM)), full((1, M_DIM)),
            full((M_DIM, M_DIM)), full((1, M_DIM)),
            full((M_DIM, 1)), full((1, 1)),
        ],
        out_specs=pl.BlockSpec((EB, SW), lambda i: (i, 0)),
        out_shape=jax.ShapeDtypeStruct((E_SL, SW), jnp.float32),
        compiler_params=pltpu.CompilerParams(
            dimension_semantics=("arbitrary",)),
    )(gbuf, ea, wr, wea, be1, we2, be2, wc1, bc1, wc2, bc2)


def _tc_prep_body(x_ref, wd_ref, ws_ref, a_ref, b_ref):
    x = x_ref[...]
    coors = x[:, 0:3]
    feats = x[:, 3:3 + D]
    pd = jnp.dot(feats, wd_ref[...], preferred_element_type=jnp.float32)
    ps = jnp.dot(feats, ws_ref[...], preferred_element_type=jnp.float32)
    pad = jnp.zeros((x.shape[0], TWT - 3 - M_DIM), jnp.float32)
    a_ref[...] = jnp.concatenate([coors, pd, pad], axis=1)
    b_ref[...] = jnp.concatenate([-coors, ps, pad], axis=1)


def _tc_prep(x, wd, ws):
    grid = (N_PAD // NB,)
    full = lambda shape: pl.BlockSpec(shape, lambda i: (0, 0))
    return pl.pallas_call(
        _tc_prep_body,
        grid=grid,
        in_specs=[
            pl.BlockSpec((NB, D + 4), lambda i: (i, 0)),
            full((D, M_DIM)), full((D, M_DIM)),
        ],
        out_specs=[pl.BlockSpec((NB, TWT), lambda i: (i, 0))] * 2,
        out_shape=[jax.ShapeDtypeStruct((N_PAD, TWT), jnp.float32)] * 2,
        compiler_params=pltpu.CompilerParams(
            dimension_semantics=("arbitrary",)),
    )(x, wd, ws)


def _node_update(x_ref, p_refs, wn1f_ref, wn1m_ref, bn1_ref,
                 wn2_ref, bn2_ref):
    x = x_ref[...]
    coors = x[:, 0:3]
    feats = x[:, 3:3 + D]
    acc = p_refs[0][...]
    for pr in p_refs[1:]:
        acc = acc + pr[...]
    m_i = acc[:, 0:M_DIM]
    num = acc[:, M_DIM:M_DIM + 3]
    cnt = acc[:, M_DIM + 3:M_DIM + 4]
    h1 = jax.nn.silu(
        jnp.dot(feats, wn1f_ref[...], preferred_element_type=jnp.float32)
        + jnp.dot(m_i, wn1m_ref[...], preferred_element_type=jnp.float32)
        + bn1_ref[...])
    fh = jnp.dot(h1, wn2_ref[...], preferred_element_type=jnp.float32) + bn2_ref[...]
    coors2 = 2.0 * coors + num / jnp.maximum(cnt, 1.0)
    feats2 = 2.0 * feats + fh
    return coors2, feats2


_NPART = H * NC


def _tc_node_body(*refs):
    x_ref = refs[0]
    p_refs = refs[1:1 + _NPART]
    (wn1f_ref, wn1m_ref, bn1_ref, wn2_ref, bn2_ref, wdn_ref, wsn_ref,
     xo_ref, a_ref, b_ref) = refs[1 + _NPART:]
    coors2, feats2 = _node_update(x_ref, p_refs, wn1f_ref, wn1m_ref,
                                  bn1_ref, wn2_ref, bn2_ref)
    pad1 = jnp.zeros((coors2.shape[0], 1), jnp.float32)
    xo_ref[...] = jnp.concatenate([coors2, feats2, pad1], axis=1)
    pd = jnp.dot(feats2, wdn_ref[...], preferred_element_type=jnp.float32)
    ps = jnp.dot(feats2, wsn_ref[...], preferred_element_type=jnp.float32)
    pad = jnp.zeros((coors2.shape[0], TWT - 3 - M_DIM), jnp.float32)
    a_ref[...] = jnp.concatenate([coors2, pd, pad], axis=1)
    b_ref[...] = jnp.concatenate([-coors2, ps, pad], axis=1)


def _tc_node(x, parts, wn1f, wn1m, bn1, wn2, bn2, wdn, wsn):
    grid = (N_PAD // NB,)
    full = lambda shape: pl.BlockSpec(shape, lambda i: (0, 0))
    return pl.pallas_call(
        _tc_node_body,
        grid=grid,
        in_specs=[
            pl.BlockSpec((NB, D + 4), lambda i: (i, 0)),
            *[pl.BlockSpec((NB, SW), lambda i: (i, 0))] * _NPART,
            full((D, M_DIM)), full((M_DIM, M_DIM)), full((1, M_DIM)),
            full((M_DIM, D)), full((1, D)),
            full((D, M_DIM)), full((D, M_DIM)),
        ],
        out_specs=[
            pl.BlockSpec((NB, D + 4), lambda i: (i, 0)),
            pl.BlockSpec((NB, TWT), lambda i: (i, 0)),
            pl.BlockSpec((NB, TWT), lambda i: (i, 0)),
        ],
        out_shape=[
            jax.ShapeDtypeStruct((N_PAD, D + 4), jnp.float32),
            jax.ShapeDtypeStruct((N_PAD, TWT), jnp.float32),
            jax.ShapeDtypeStruct((N_PAD, TWT), jnp.float32),
        ],
        compiler_params=pltpu.CompilerParams(
            dimension_semantics=("arbitrary",)),
    )(x, *parts, wn1f, wn1m, bn1, wn2, bn2, wdn, wsn)


def _tc_head_body(*refs):
    x_ref = refs[0]
    p_refs = refs[1:1 + _NPART]
    (wn1f_ref, wn1m_ref, bn1_ref, wn2_ref, bn2_ref,
     wh1_ref, bh1_ref, wh2_ref, bh2_ref, o_ref) = refs[1 + _NPART:]
    _, feats2 = _node_update(x_ref, p_refs, wn1f_ref, wn1m_ref,
                             bn1_ref, wn2_ref, bn2_ref)
    hh = jax.nn.silu(
        jnp.dot(feats2, wh1_ref[...], preferred_element_type=jnp.float32)
        + bh1_ref[...])
    o_ref[...] = (jnp.dot(hh, wh2_ref[...], preferred_element_type=jnp.float32)
                  + bh2_ref[...])


def _tc_head(x, parts, wn1f, wn1m, bn1, wn2, bn2, wh1, bh1, wh2, bh2):
    grid = (N_PAD // NB,)
    full = lambda shape: pl.BlockSpec(shape, lambda i: (0, 0))
    return pl.pallas_call(
        _tc_head_body,
        grid=grid,
        in_specs=[
            pl.BlockSpec((NB, D + 4), lambda i: (i, 0)),
            *[pl.BlockSpec((NB, SW), lambda i: (i, 0))] * _NPART,
            full((D, M_DIM)), full((M_DIM, M_DIM)), full((1, M_DIM)),
            full((M_DIM, D)), full((1, D)),
            full((D, 16)), full((1, 16)), full((16, 2)), full((1, 2)),
        ],
        out_specs=pl.BlockSpec((NB, 2), lambda i: (i, 0)),
        out_shape=jax.ShapeDtypeStruct((N_PAD, 2), jnp.float32),
        compiler_params=pltpu.CompilerParams(
            dimension_semantics=("arbitrary",)),
    )(x, *parts, wn1f, wn1m, bn1, wn2, bn2, wh1, bh1, wh2, bh2)


def kernel(x, pos, edge_index, edge_attr, batch, esm_rep, prop, params):
    del x, batch, esm_rep
    f32 = jnp.float32

    src = edge_index[0]
    dst = edge_index[1]
    pad_e = E_PAD - E
    src_g = jnp.concatenate([src, jnp.zeros((pad_e,), jnp.int32)])
    dst_g = jnp.concatenate([dst, jnp.zeros((pad_e,), jnp.int32)])
    dst_s = jnp.concatenate([dst, jnp.full((pad_e,), N, jnp.int32)])
    src_g = src_g.reshape(H, NW, NCHUNK, CH)
    dst_g = dst_g.reshape(H, NW, NCHUNK, CH)
    dst_s = dst_s.reshape(H, NW, NCHUNK, CH)
    ea = jnp.concatenate([edge_attr,
                          jnp.zeros((pad_e, EDGE_DIM), f32)], axis=0)
    ea = ea.reshape(H, E_SL, EDGE_DIM)

    x0 = jnp.concatenate(
        [pos, prop, jnp.zeros((N, 1), f32)], axis=1)
    x0 = jnp.concatenate(
        [x0, jnp.zeros((N_PAD - N, D + 4), f32)], axis=0)
    zeros_acc = jnp.zeros((N_PAD, SW), f32)

    layers = params["layers"]

    def split_we1(p):
        we1 = p["We1"]
        return (we1[0:D], we1[D:2 * D], we1[2 * D:2 * D + 1],
                we1[2 * D + 1:])

    wd0, ws0, _, _ = split_we1(layers[0])
    a_t, b_t = _tc_prep(x0, wd0, ws0)

    xc = x0
    for l, p in enumerate(layers):
        _, _, wr, wea = split_we1(p)
        parts = []
        for h in range(H):
            gbuf = _sc_gather(a_t, b_t, dst_g[h], src_g[h])
            sbuf = _tc_edge(gbuf, ea[h], wr, wea,
                            p["be1"].reshape(1, M_DIM), p["We2"],
                            p["be2"].reshape(1, M_DIM), p["Wc1"],
                            p["bc1"].reshape(1, M_DIM), p["Wc2"],
                            p["bc2"].reshape(1, 1))
            part = _sc_scatter(sbuf, dst_s[h], zeros_acc)
            parts.append(part[0])
            parts.append(part[1])
        wn1 = p["Wn1"]
        if l < N_LAYERS - 1:
            wdn, wsn, _, _ = split_we1(layers[l + 1])
            xc, a_t, b_t = _tc_node(
                xc, parts, wn1[0:D], wn1[D:],
                p["bn1"].reshape(1, M_DIM), p["Wn2"],
                p["bn2"].reshape(1, D), wdn, wsn)
        else:
            hd = params["head"]
            out = _tc_head(
                xc, parts, wn1[0:D], wn1[D:],
                p["bn1"].reshape(1, M_DIM), p["Wn2"],
                p["bn2"].reshape(1, D),
                hd["W1"], hd["b1"].reshape(1, 16),
                hd["W2"], hd["b2"].reshape(1, 2))
    return out[:N]


# speedup vs baseline: 1.0401x; 1.0401x over previous
import jax
import jax.numpy as jnp
from jax import lax
from jax.experimental import pallas as pl
from jax.experimental.pallas import tpu as pltpu
from jax.experimental.pallas import tpu_sc as plsc

N = 10000
E = 320000
D = 128
EDGE_DIM = 16
M_DIM = 64
N_LAYERS = 3

NC = 2
NS = 16
NW = NC * NS

CH = 128
H = 4
NCHUNK = 20
PT = CH * NCHUNK
E_SL = PT * NW
E_PAD = E_SL * H
N_PAD = 10240
RPT = N_PAD // NS

TWT = 128
SW = 128

EB = 2048
NB = 512

_MESH = plsc.VectorSubcoreMesh(
    core_axis_name="c", subcore_axis_name="s", num_cores=NC, num_subcores=NS)


def _sc_gather_body(a_hbm, b_hbm, dst_hbm, src_hbm, g_hbm,
                    idx_d, idx_s, buf_a0, buf_b0, buf_a1, buf_b1,
                    sem_a0, sem_b0, sem_a1, sem_b1):
    c = lax.axis_index("c")
    s = lax.axis_index("s")
    wid = s * NC + c
    ebase = wid * PT

    pltpu.sync_copy(dst_hbm.at[wid], idx_d)
    pltpu.sync_copy(src_hbm.at[wid], idx_s)

    slots = ((buf_a0, buf_b0, sem_a0, sem_b0),
             (buf_a1, buf_b1, sem_a1, sem_b1))

    def issue(slot, k):
        ba, bb, sa, sb = slots[slot]
        pltpu.async_copy(a_hbm.at[idx_d.at[k]], ba, sa)
        pltpu.async_copy(b_hbm.at[idx_s.at[k]], bb, sb)

    def process(slot, k):
        ba, bb, sa, sb = slots[slot]
        pltpu.make_async_copy(a_hbm.at[idx_d.at[k]], ba, sa).wait()
        pltpu.make_async_copy(b_hbm.at[idx_s.at[k]], bb, sb).wait()

        @pl.loop(0, CH, unroll=4)
        def addrow(r):
            for j in range(5):
                sl = pl.ds(j * 16, 16)
                ba[r, sl] = ba[r, sl] + bb[r, sl]

        pltpu.sync_copy(ba, g_hbm.at[pl.ds(ebase + k * CH, CH)])

    issue(0, 0)

    @pl.loop(0, (NCHUNK - 2) // 2)
    def it(j):
        k0 = 2 * j
        issue(1, k0 + 1)
        process(0, k0)
        issue(0, k0 + 2)
        process(1, k0 + 1)

    issue(1, NCHUNK - 1)
    process(0, NCHUNK - 2)
    process(1, NCHUNK - 1)


_sc_gather = pl.kernel(
    _sc_gather_body,
    out_type=jax.ShapeDtypeStruct((E_SL, TWT), jnp.float32),
    mesh=_MESH,
    scratch_types=[
        pltpu.VMEM((NCHUNK, CH), jnp.int32),
        pltpu.VMEM((NCHUNK, CH), jnp.int32),
        pltpu.VMEM((CH, TWT), jnp.float32),
        pltpu.VMEM((CH, TWT), jnp.float32),
        pltpu.VMEM((CH, TWT), jnp.float32),
        pltpu.VMEM((CH, TWT), jnp.float32),
        pltpu.SemaphoreType.DMA,
        pltpu.SemaphoreType.DMA,
        pltpu.SemaphoreType.DMA,
        pltpu.SemaphoreType.DMA,
    ],
)


def _sc_scatter_body(s_hbm, dst_hbm, zero_hbm, p_hbm, idx, buf0, buf1,
                     sem0, sem1, acc):
    c = lax.axis_index("c")
    s = lax.axis_index("s")
    wid = s * NC + c
    ebase = wid * PT

    pltpu.sync_copy(zero_hbm.at[pl.ds(s * RPT, RPT)], acc.at[pl.ds(s * RPT, RPT)])
    pltpu.sync_copy(dst_hbm.at[wid], idx)
    plsc.subcore_barrier()

    slots = ((buf0, sem0), (buf1, sem1))

    def issue(slot, k):
        b, sem = slots[slot]
        pltpu.async_copy(s_hbm.at[pl.ds(ebase + k * CH, CH)], b, sem)

    def process(slot, k):
        b, sem = slots[slot]
        pltpu.make_async_copy(s_hbm.at[pl.ds(ebase + k * CH, CH)], b, sem).wait()
        pltpu.sync_copy(b, acc.at[idx.at[k]], add=True)

    issue(0, 0)

    @pl.loop(0, (NCHUNK - 2) // 2)
    def it(j):
        k0 = 2 * j
        issue(1, k0 + 1)
        process(0, k0)
        issue(0, k0 + 2)
        process(1, k0 + 1)

    issue(1, NCHUNK - 1)
    process(0, NCHUNK - 2)
    process(1, NCHUNK - 1)

    plsc.subcore_barrier()
    pltpu.sync_copy(acc.at[pl.ds(s * RPT, RPT)], p_hbm.at[c, pl.ds(s * RPT, RPT)])


_sc_scatter = pl.kernel(
    _sc_scatter_body,
    out_type=jax.ShapeDtypeStruct((NC, N_PAD, SW), jnp.float32),
    mesh=_MESH,
    scratch_types=[
        pltpu.VMEM((NCHUNK, CH), jnp.int32),
        pltpu.VMEM((CH, SW), jnp.float32),
        pltpu.VMEM((CH, SW), jnp.float32),
        pltpu.SemaphoreType.DMA,
        pltpu.SemaphoreType.DMA,
        pltpu.MemorySpace.VMEM_SHARED((N_PAD, SW), jnp.float32),
    ],
)


def _tc_edge_body(g_ref, ea_ref, wr_ref, wea_ref, be1_ref, we2_ref, be2_ref,
                  wc1_ref, bc1_ref, wc2_ref, bc2_ref, s_ref):
    g = g_ref[...]
    rel = g[:, 0:3]
    q = g[:, 3:3 + M_DIM]
    rd = jnp.sum(rel * rel, axis=1, keepdims=True)
    pre = (q + rd * wr_ref[...]
           + jnp.dot(ea_ref[...], wea_ref[...], preferred_element_type=jnp.float32)
           + be1_ref[...])
    m = jax.nn.silu(pre)
    m = jax.nn.silu(jnp.dot(m, we2_ref[...], preferred_element_type=jnp.float32)
                    + be2_ref[...])
    cwh = jax.nn.silu(jnp.dot(m, wc1_ref[...], preferred_element_type=jnp.float32)
                      + bc1_ref[...])
    cw = jnp.dot(cwh, wc2_ref[...], preferred_element_type=jnp.float32) + bc2_ref[...]
    ones = jnp.ones((m.shape[0], 1), jnp.float32)
    pad = jnp.zeros((m.shape[0], SW - M_DIM - 4), jnp.float32)
    s_ref[...] = jnp.concatenate([m, cw * rel, ones, pad], axis=1)


def _tc_edge(gbuf, ea, wr, wea, be1, we2, be2, wc1, bc1, wc2, bc2):
    grid = (E_SL // EB,)
    full = lambda shape: pl.BlockSpec(shape, lambda i: (0, 0))
    return pl.pallas_call(
        _tc_edge_body,
        grid=grid,
        in_specs=[
            pl.BlockSpec((EB, TWT), lambda i: (i, 0)),
            pl.BlockSpec((EB, EDGE_DIM), lambda i: (i, 0)),
            full((1, M_DIM)), full((EDGE_DIM, M_DIM)), full((1, M_DIM)),
            full((M_DIM, M_DIM)), full((1, M_DIM)),
            full((M_DIM, M_DIM)), full((1, M_DIM)),
            full((M_DIM, 1)), full((1, 1)),
        ],
        out_specs=pl.BlockSpec((EB, SW), lambda i: (i, 0)),
        out_shape=jax.ShapeDtypeStruct((E_SL, SW), jnp.float32),
        compiler_params=pltpu.CompilerParams(
            dimension_semantics=("arbitrary",)),
    )(gbuf, ea, wr, wea, be1, we2, be2, wc1, bc1, wc2, bc2)


def _tc_prep_body(x_ref, wd_ref, ws_ref, a_ref, b_ref):
    x = x_ref[...]
    coors = x[:, 0:3]
    feats = x[:, 3:3 + D]
    pd = jnp.dot(feats, wd_ref[...], preferred_element_type=jnp.float32)
    ps = jnp.dot(feats, ws_ref[...], preferred_element_type=jnp.float32)
    pad = jnp.zeros((x.shape[0], TWT - 3 - M_DIM), jnp.float32)
    a_ref[...] = jnp.concatenate([coors, pd, pad], axis=1)
    b_ref[...] = jnp.concatenate([-coors, ps, pad], axis=1)


def _tc_prep(x, wd, ws):
    grid = (N_PAD // NB,)
    full = lambda shape: pl.BlockSpec(shape, lambda i: (0, 0))
    return pl.pallas_call(
        _tc_prep_body,
        grid=grid,
        in_specs=[
            pl.BlockSpec((NB, D + 4), lambda i: (i, 0)),
            full((D, M_DIM)), full((D, M_DIM)),
        ],
        out_specs=[pl.BlockSpec((NB, TWT), lambda i: (i, 0))] * 2,
        out_shape=[jax.ShapeDtypeStruct((N_PAD, TWT), jnp.float32)] * 2,
        compiler_params=pltpu.CompilerParams(
            dimension_semantics=("arbitrary",)),
    )(x, wd, ws)


def _node_update(x_ref, p_refs, wn1f_ref, wn1m_ref, bn1_ref,
                 wn2_ref, bn2_ref):
    x = x_ref[...]
    coors = x[:, 0:3]
    feats = x[:, 3:3 + D]
    acc = p_refs[0][...]
    for pr in p_refs[1:]:
        acc = acc + pr[...]
    m_i = acc[:, 0:M_DIM]
    num = acc[:, M_DIM:M_DIM + 3]
    cnt = acc[:, M_DIM + 3:M_DIM + 4]
    h1 = jax.nn.silu(
        jnp.dot(feats, wn1f_ref[...], preferred_element_type=jnp.float32)
        + jnp.dot(m_i, wn1m_ref[...], preferred_element_type=jnp.float32)
        + bn1_ref[...])
    fh = jnp.dot(h1, wn2_ref[...], preferred_element_type=jnp.float32) + bn2_ref[...]
    coors2 = 2.0 * coors + num / jnp.maximum(cnt, 1.0)
    feats2 = 2.0 * feats + fh
    return coors2, feats2


_NPART = H * NC


def _tc_node_body(*refs):
    x_ref = refs[0]
    p_refs = refs[1:1 + _NPART]
    (wn1f_ref, wn1m_ref, bn1_ref, wn2_ref, bn2_ref, wdn_ref, wsn_ref,
     xo_ref, a_ref, b_ref) = refs[1 + _NPART:]
    coors2, feats2 = _node_update(x_ref, p_refs, wn1f_ref, wn1m_ref,
                                  bn1_ref, wn2_ref, bn2_ref)
    pad1 = jnp.zeros((coors2.shape[0], 1), jnp.float32)
    xo_ref[...] = jnp.concatenate([coors2, feats2, pad1], axis=1)
    pd = jnp.dot(feats2, wdn_ref[...], preferred_element_type=jnp.float32)
    ps = jnp.dot(feats2, wsn_ref[...], preferred_element_type=jnp.float32)
    pad = jnp.zeros((coors2.shape[0], TWT - 3 - M_DIM), jnp.float32)
    a_ref[...] = jnp.concatenate([coors2, pd, pad], axis=1)
    b_ref[...] = jnp.concatenate([-coors2, ps, pad], axis=1)


def _tc_node(x, parts, wn1f, wn1m, bn1, wn2, bn2, wdn, wsn):
    grid = (N_PAD // NB,)
    full = lambda shape: pl.BlockSpec(shape, lambda i: (0, 0))
    return pl.pallas_call(
        _tc_node_body,
        grid=grid,
        in_specs=[
            pl.BlockSpec((NB, D + 4), lambda i: (i, 0)),
            *[pl.BlockSpec((NB, SW), lambda i: (i, 0))] * _NPART,
            full((D, M_DIM)), full((M_DIM, M_DIM)), full((1, M_DIM)),
            full((M_DIM, D)), full((1, D)),
            full((D, M_DIM)), full((D, M_DIM)),
        ],
        out_specs=[
            pl.BlockSpec((NB, D + 4), lambda i: (i, 0)),
            pl.BlockSpec((NB, TWT), lambda i: (i, 0)),
            pl.BlockSpec((NB, TWT), lambda i: (i, 0)),
        ],
        out_shape=[
            jax.ShapeDtypeStruct((N_PAD, D + 4), jnp.float32),
            jax.ShapeDtypeStruct((N_PAD, TWT), jnp.float32),
            jax.ShapeDtypeStruct((N_PAD, TWT), jnp.float32),
        ],
        compiler_params=pltpu.CompilerParams(
            dimension_semantics=("arbitrary",)),
    )(x, *parts, wn1f, wn1m, bn1, wn2, bn2, wdn, wsn)


def _tc_head_body(*refs):
    x_ref = refs[0]
    p_refs = refs[1:1 + _NPART]
    (wn1f_ref, wn1m_ref, bn1_ref, wn2_ref, bn2_ref,
     wh1_ref, bh1_ref, wh2_ref, bh2_ref, o_ref) = refs[1 + _NPART:]
    _, feats2 = _node_update(x_ref, p_refs, wn1f_ref, wn1m_ref,
                             bn1_ref, wn2_ref, bn2_ref)
    hh = jax.nn.silu(
        jnp.dot(feats2, wh1_ref[...], preferred_element_type=jnp.float32)
        + bh1_ref[...])
    o_ref[...] = (jnp.dot(hh, wh2_ref[...], preferred_element_type=jnp.float32)
                  + bh2_ref[...])


def _tc_head(x, parts, wn1f, wn1m, bn1, wn2, bn2, wh1, bh1, wh2, bh2):
    grid = (N_PAD // NB,)
    full = lambda shape: pl.BlockSpec(shape, lambda i: (0, 0))
    return pl.pallas_call(
        _tc_head_body,
        grid=grid,
        in_specs=[
            pl.BlockSpec((NB, D + 4), lambda i: (i, 0)),
            *[pl.BlockSpec((NB, SW), lambda i: (i, 0))] * _NPART,
            full((D, M_DIM)), full((M_DIM, M_DIM)), full((1, M_DIM)),
            full((M_DIM, D)), full((1, D)),
            full((D, 16)), full((1, 16)), full((16, 2)), full((1, 2)),
        ],
        out_specs=pl.BlockSpec((NB, 2), lambda i: (i, 0)),
        out_shape=jax.ShapeDtypeStruct((N_PAD, 2), jnp.float32),
        compiler_params=pltpu.CompilerParams(
            dimension_semantics=("arbitrary",)),
    )(x, *parts, wn1f, wn1m, bn1, wn2, bn2, wh1, bh1, wh2, bh2)


def kernel(x, pos, edge_index, edge_attr, batch, esm_rep, prop, params):
    del x, batch, esm_rep
    f32 = jnp.float32

    src = edge_index[0]
    dst = edge_index[1]
    pad_e = E_PAD - E
    src_g = jnp.concatenate([src, jnp.zeros((pad_e,), jnp.int32)])
    dst_g = jnp.concatenate([dst, jnp.zeros((pad_e,), jnp.int32)])
    dst_s = jnp.concatenate([dst, jnp.full((pad_e,), N, jnp.int32)])
    src_g = src_g.reshape(H, NW, NCHUNK, CH)
    dst_g = dst_g.reshape(H, NW, NCHUNK, CH)
    dst_s = dst_s.reshape(H, NW, NCHUNK, CH)
    ea = jnp.concatenate([edge_attr,
                          jnp.zeros((pad_e, EDGE_DIM), f32)], axis=0)
    ea = ea.reshape(H, E_SL, EDGE_DIM)

    x0 = jnp.concatenate(
        [pos, prop, jnp.zeros((N, 1), f32)], axis=1)
    x0 = jnp.concatenate(
        [x0, jnp.zeros((N_PAD - N, D + 4), f32)], axis=0)
    zeros_acc = jnp.zeros((N_PAD, SW), f32)

    layers = params["layers"]

    def split_we1(p):
        we1 = p["We1"]
        return (we1[0:D], we1[D:2 * D], we1[2 * D:2 * D + 1],
                we1[2 * D + 1:])

    wd0, ws0, _, _ = split_we1(layers[0])
    a_t, b_t = _tc_prep(x0, wd0, ws0)

    xc = x0
    for l, p in enumerate(layers):
        _, _, wr, wea = split_we1(p)
        parts = []
        for h in range(H):
            gbuf = _sc_gather(a_t, b_t, dst_g[h], src_g[h])
            sbuf = _tc_edge(gbuf, ea[h], wr, wea,
                            p["be1"].reshape(1, M_DIM), p["We2"],
                            p["be2"].reshape(1, M_DIM), p["Wc1"],
                            p["bc1"].reshape(1, M_DIM), p["Wc2"],
                            p["bc2"].reshape(1, 1))
            part = _sc_scatter(sbuf, dst_s[h], zeros_acc)
            parts.append(part[0])
            parts.append(part[1])
        wn1 = p["Wn1"]
        if l < N_LAYERS - 1:
            wdn, wsn, _, _ = split_we1(layers[l + 1])
            xc, a_t, b_t = _tc_node(
                xc, parts, wn1[0:D], wn1[D:],
                p["bn1"].reshape(1, M_DIM), p["Wn2"],
                p["bn2"].reshape(1, D), wdn, wsn)
        else:
            hd = params["head"]
            out = _tc_head(
                xc, parts, wn1[0:D], wn1[D:],
                p["bn1"].reshape(1, M_DIM), p["Wn2"],
                p["bn2"].reshape(1, D),
                hd["W1"], hd["b1"].reshape(1, 16),
                hd["W2"], hd["b2"].reshape(1, 2))
    return out[:N]


# speedup vs baseline: 1.0805x; 1.0389x over previous
import jax
import jax.numpy as jnp
from jax import lax
from jax.experimental import pallas as pl
from jax.experimental.pallas import tpu as pltpu
from jax.experimental.pallas import tpu_sc as plsc

N = 10000
E = 320000
D = 128
EDGE_DIM = 16
M_DIM = 64
N_LAYERS = 3

NC = 2
NS = 16
NW = NC * NS

CH = 128
H = 4
NCHUNK = 20
PT = CH * NCHUNK
E_SL = PT * NW
E_PAD = E_SL * H
N_PAD = 10240
RPT = N_PAD // NS

TWT = 128
SW = 128

EB = 2048
NB = 512

_MESH = plsc.VectorSubcoreMesh(
    core_axis_name="c", subcore_axis_name="s", num_cores=NC, num_subcores=NS)


def _sc_gather_body(a_hbm, b_hbm, dst_hbm, src_hbm, g_hbm,
                    idx_d, idx_s, buf_a0, buf_b0, buf_a1, buf_b1,
                    sem_a0, sem_b0, sem_a1, sem_b1):
    c = lax.axis_index("c")
    s = lax.axis_index("s")
    wid = s * NC + c
    ebase = wid * PT

    pltpu.sync_copy(dst_hbm.at[wid], idx_d)
    pltpu.sync_copy(src_hbm.at[wid], idx_s)

    slots = ((buf_a0, buf_b0, sem_a0, sem_b0),
             (buf_a1, buf_b1, sem_a1, sem_b1))

    def issue(slot, k):
        ba, bb, sa, sb = slots[slot]
        pltpu.async_copy(a_hbm.at[idx_d.at[k]], ba, sa)
        pltpu.async_copy(b_hbm.at[idx_s.at[k]], bb, sb)

    def process(slot, k):
        ba, bb, sa, sb = slots[slot]
        pltpu.make_async_copy(a_hbm.at[idx_d.at[k]], ba, sa).wait()
        pltpu.make_async_copy(b_hbm.at[idx_s.at[k]], bb, sb).wait()

        @pl.loop(0, CH, unroll=4)
        def addrow(r):
            for j in range(5):
                sl = pl.ds(j * 16, 16)
                ba[r, sl] = ba[r, sl] + bb[r, sl]

        pltpu.sync_copy(ba, g_hbm.at[pl.ds(ebase + k * CH, CH)])

    issue(0, 0)

    @pl.loop(0, (NCHUNK - 2) // 2)
    def it(j):
        k0 = 2 * j
        issue(1, k0 + 1)
        process(0, k0)
        issue(0, k0 + 2)
        process(1, k0 + 1)

    issue(1, NCHUNK - 1)
    process(0, NCHUNK - 2)
    process(1, NCHUNK - 1)


_sc_gather = pl.kernel(
    _sc_gather_body,
    out_type=jax.ShapeDtypeStruct((E_SL, TWT), jnp.float32),
    mesh=_MESH,
    scratch_types=[
        pltpu.VMEM((NCHUNK, CH), jnp.int32),
        pltpu.VMEM((NCHUNK, CH), jnp.int32),
        pltpu.VMEM((CH, TWT), jnp.float32),
        pltpu.VMEM((CH, TWT), jnp.float32),
        pltpu.VMEM((CH, TWT), jnp.float32),
        pltpu.VMEM((CH, TWT), jnp.float32),
        pltpu.SemaphoreType.DMA,
        pltpu.SemaphoreType.DMA,
        pltpu.SemaphoreType.DMA,
        pltpu.SemaphoreType.DMA,
    ],
)


def _sc_scatter_body(s_hbm, dst_hbm, zero_hbm, p_hbm, idx, buf0, buf1,
                     sem0, sem1, acc):
    c = lax.axis_index("c")
    s = lax.axis_index("s")
    wid = s * NC + c
    ebase = wid * PT

    pltpu.sync_copy(zero_hbm.at[pl.ds(s * RPT, RPT)], acc.at[pl.ds(s * RPT, RPT)])
    pltpu.sync_copy(dst_hbm.at[wid], idx)
    plsc.subcore_barrier()

    slots = ((buf0, sem0), (buf1, sem1))

    def issue(slot, k):
        b, sem = slots[slot]
        pltpu.async_copy(s_hbm.at[pl.ds(ebase + k * CH, CH)], b, sem)

    def process(slot, k):
        b, sem = slots[slot]
        pltpu.make_async_copy(s_hbm.at[pl.ds(ebase + k * CH, CH)], b, sem).wait()
        pltpu.sync_copy(b, acc.at[idx.at[k]], add=True)

    issue(0, 0)

    @pl.loop(0, (NCHUNK - 2) // 2)
    def it(j):
        k0 = 2 * j
        issue(1, k0 + 1)
        process(0, k0)
        issue(0, k0 + 2)
        process(1, k0 + 1)

    issue(1, NCHUNK - 1)
    process(0, NCHUNK - 2)
    process(1, NCHUNK - 1)

    plsc.subcore_barrier()
    pltpu.sync_copy(acc.at[pl.ds(s * RPT, RPT)], p_hbm.at[c, pl.ds(s * RPT, RPT)])


_sc_scatter = pl.kernel(
    _sc_scatter_body,
    out_type=jax.ShapeDtypeStruct((NC, N_PAD, SW), jnp.float32),
    mesh=_MESH,
    scratch_types=[
        pltpu.VMEM((NCHUNK, CH), jnp.int32),
        pltpu.VMEM((CH, SW), jnp.float32),
        pltpu.VMEM((CH, SW), jnp.float32),
        pltpu.SemaphoreType.DMA,
        pltpu.SemaphoreType.DMA,
        pltpu.MemorySpace.VMEM_SHARED((N_PAD, SW), jnp.float32),
    ],
)


def _tc_edge_body(g_ref, ea_ref, wr_ref, wea_ref, be1_ref, we2_ref, be2_ref,
                  wc1_ref, bc1_ref, wc2_ref, bc2_ref, s_ref):
    g = g_ref[...]
    rel = g[:, 0:3]
    q = g[:, 3:3 + M_DIM]
    rd = jnp.sum(rel * rel, axis=1, keepdims=True)
    eap = lax.dot_general(ea_ref[...], wea_ref[...],
                          (((0,), (0,)), ((), ())),
                          preferred_element_type=jnp.float32)
    pre = (q + rd * wr_ref[...]
           + eap
           + be1_ref[...])
    m = jax.nn.silu(pre)
    m = jax.nn.silu(jnp.dot(m, we2_ref[...], preferred_element_type=jnp.float32)
                    + be2_ref[...])
    cwh = jax.nn.silu(jnp.dot(m, wc1_ref[...], preferred_element_type=jnp.float32)
                      + bc1_ref[...])
    cw = jnp.dot(cwh, wc2_ref[...], preferred_element_type=jnp.float32) + bc2_ref[...]
    ones = jnp.ones((m.shape[0], 1), jnp.float32)
    pad = jnp.zeros((m.shape[0], SW - M_DIM - 4), jnp.float32)
    s_ref[...] = jnp.concatenate([m, cw * rel, ones, pad], axis=1)


def _tc_edge(gbuf, ea_t, h_off, wr, wea, be1, we2, be2, wc1, bc1, wc2, bc2):
    grid = (E_SL // EB,)
    full = lambda shape: pl.BlockSpec(shape, lambda i: (0, 0))
    return pl.pallas_call(
        _tc_edge_body,
        grid=grid,
        in_specs=[
            pl.BlockSpec((EB, TWT), lambda i: (i, 0)),
            pl.BlockSpec((EDGE_DIM, EB), lambda i, o=h_off: (0, i + o)),
            full((1, M_DIM)), full((EDGE_DIM, M_DIM)), full((1, M_DIM)),
            full((M_DIM, M_DIM)), full((1, M_DIM)),
            full((M_DIM, M_DIM)), full((1, M_DIM)),
            full((M_DIM, 1)), full((1, 1)),
        ],
        out_specs=pl.BlockSpec((EB, SW), lambda i: (i, 0)),
        out_shape=jax.ShapeDtypeStruct((E_SL, SW), jnp.float32),
        compiler_params=pltpu.CompilerParams(
            dimension_semantics=("arbitrary",)),
    )(gbuf, ea_t, wr, wea, be1, we2, be2, wc1, bc1, wc2, bc2)


def _tc_prep_body(x_ref, wd_ref, ws_ref, a_ref, b_ref):
    x = x_ref[...]
    coors = x[:, 0:3]
    feats = x[:, 3:3 + D]
    pd = jnp.dot(feats, wd_ref[...], preferred_element_type=jnp.float32)
    ps = jnp.dot(feats, ws_ref[...], preferred_element_type=jnp.float32)
    pad = jnp.zeros((x.shape[0], TWT - 3 - M_DIM), jnp.float32)
    a_ref[...] = jnp.concatenate([coors, pd, pad], axis=1)
    b_ref[...] = jnp.concatenate([-coors, ps, pad], axis=1)


def _tc_prep(x, wd, ws):
    grid = (N_PAD // NB,)
    full = lambda shape: pl.BlockSpec(shape, lambda i: (0, 0))
    return pl.pallas_call(
        _tc_prep_body,
        grid=grid,
        in_specs=[
            pl.BlockSpec((NB, D + 4), lambda i: (i, 0)),
            full((D, M_DIM)), full((D, M_DIM)),
        ],
        out_specs=[pl.BlockSpec((NB, TWT), lambda i: (i, 0))] * 2,
        out_shape=[jax.ShapeDtypeStruct((N_PAD, TWT), jnp.float32)] * 2,
        compiler_params=pltpu.CompilerParams(
            dimension_semantics=("arbitrary",)),
    )(x, wd, ws)


def _node_update(x_ref, p_refs, wn1f_ref, wn1m_ref, bn1_ref,
                 wn2_ref, bn2_ref):
    x = x_ref[...]
    coors = x[:, 0:3]
    feats = x[:, 3:3 + D]
    acc = p_refs[0][...]
    for pr in p_refs[1:]:
        acc = acc + pr[...]
    m_i = acc[:, 0:M_DIM]
    num = acc[:, M_DIM:M_DIM + 3]
    cnt = acc[:, M_DIM + 3:M_DIM + 4]
    h1 = jax.nn.silu(
        jnp.dot(feats, wn1f_ref[...], preferred_element_type=jnp.float32)
        + jnp.dot(m_i, wn1m_ref[...], preferred_element_type=jnp.float32)
        + bn1_ref[...])
    fh = jnp.dot(h1, wn2_ref[...], preferred_element_type=jnp.float32) + bn2_ref[...]
    coors2 = 2.0 * coors + num / jnp.maximum(cnt, 1.0)
    feats2 = 2.0 * feats + fh
    return coors2, feats2


_NPART = H * NC


def _tc_node_body(*refs):
    x_ref = refs[0]
    p_refs = refs[1:1 + _NPART]
    (wn1f_ref, wn1m_ref, bn1_ref, wn2_ref, bn2_ref, wdn_ref, wsn_ref,
     xo_ref, a_ref, b_ref) = refs[1 + _NPART:]
    coors2, feats2 = _node_update(x_ref, p_refs, wn1f_ref, wn1m_ref,
                                  bn1_ref, wn2_ref, bn2_ref)
    pad1 = jnp.zeros((coors2.shape[0], 1), jnp.float32)
    xo_ref[...] = jnp.concatenate([coors2, feats2, pad1], axis=1)
    pd = jnp.dot(feats2, wdn_ref[...], preferred_element_type=jnp.float32)
    ps = jnp.dot(feats2, wsn_ref[...], preferred_element_type=jnp.float32)
    pad = jnp.zeros((coors2.shape[0], TWT - 3 - M_DIM), jnp.float32)
    a_ref[...] = jnp.concatenate([coors2, pd, pad], axis=1)
    b_ref[...] = jnp.concatenate([-coors2, ps, pad], axis=1)


def _tc_node(x, parts, wn1f, wn1m, bn1, wn2, bn2, wdn, wsn):
    grid = (N_PAD // NB,)
    full = lambda shape: pl.BlockSpec(shape, lambda i: (0, 0))
    return pl.pallas_call(
        _tc_node_body,
        grid=grid,
        in_specs=[
            pl.BlockSpec((NB, D + 4), lambda i: (i, 0)),
            *[pl.BlockSpec((NB, SW), lambda i: (i, 0))] * _NPART,
            full((D, M_DIM)), full((M_DIM, M_DIM)), full((1, M_DIM)),
            full((M_DIM, D)), full((1, D)),
            full((D, M_DIM)), full((D, M_DIM)),
        ],
        out_specs=[
            pl.BlockSpec((NB, D + 4), lambda i: (i, 0)),
            pl.BlockSpec((NB, TWT), lambda i: (i, 0)),
            pl.BlockSpec((NB, TWT), lambda i: (i, 0)),
        ],
        out_shape=[
            jax.ShapeDtypeStruct((N_PAD, D + 4), jnp.float32),
            jax.ShapeDtypeStruct((N_PAD, TWT), jnp.float32),
            jax.ShapeDtypeStruct((N_PAD, TWT), jnp.float32),
        ],
        compiler_params=pltpu.CompilerParams(
            dimension_semantics=("arbitrary",)),
    )(x, *parts, wn1f, wn1m, bn1, wn2, bn2, wdn, wsn)


def _tc_head_body(*refs):
    x_ref = refs[0]
    p_refs = refs[1:1 + _NPART]
    (wn1f_ref, wn1m_ref, bn1_ref, wn2_ref, bn2_ref,
     wh1_ref, bh1_ref, wh2_ref, bh2_ref, o_ref) = refs[1 + _NPART:]
    _, feats2 = _node_update(x_ref, p_refs, wn1f_ref, wn1m_ref,
                             bn1_ref, wn2_ref, bn2_ref)
    hh = jax.nn.silu(
        jnp.dot(feats2, wh1_ref[...], preferred_element_type=jnp.float32)
        + bh1_ref[...])
    o_ref[...] = (jnp.dot(hh, wh2_ref[...], preferred_element_type=jnp.float32)
                  + bh2_ref[...])


def _tc_head(x, parts, wn1f, wn1m, bn1, wn2, bn2, wh1, bh1, wh2, bh2):
    grid = (N_PAD // NB,)
    full = lambda shape: pl.BlockSpec(shape, lambda i: (0, 0))
    return pl.pallas_call(
        _tc_head_body,
        grid=grid,
        in_specs=[
            pl.BlockSpec((NB, D + 4), lambda i: (i, 0)),
            *[pl.BlockSpec((NB, SW), lambda i: (i, 0))] * _NPART,
            full((D, M_DIM)), full((M_DIM, M_DIM)), full((1, M_DIM)),
            full((M_DIM, D)), full((1, D)),
            full((D, 16)), full((1, 16)), full((16, 2)), full((1, 2)),
        ],
        out_specs=pl.BlockSpec((NB, 2), lambda i: (i, 0)),
        out_shape=jax.ShapeDtypeStruct((N_PAD, 2), jnp.float32),
        compiler_params=pltpu.CompilerParams(
            dimension_semantics=("arbitrary",)),
    )(x, *parts, wn1f, wn1m, bn1, wn2, bn2, wh1, bh1, wh2, bh2)


def kernel(x, pos, edge_index, edge_attr, batch, esm_rep, prop, params):
    del x, batch, esm_rep
    f32 = jnp.float32

    src = edge_index[0]
    dst = edge_index[1]
    pad_e = E_PAD - E
    src_g = jnp.concatenate([src, jnp.zeros((pad_e,), jnp.int32)])
    dst_g = jnp.concatenate([dst, jnp.zeros((pad_e,), jnp.int32)])
    dst_s = jnp.concatenate([dst, jnp.full((pad_e,), N, jnp.int32)])
    src_g = src_g.reshape(H, NW, NCHUNK, CH)
    dst_g = dst_g.reshape(H, NW, NCHUNK, CH)
    dst_s = dst_s.reshape(H, NW, NCHUNK, CH)
    ea_t = jnp.transpose(jnp.concatenate(
        [edge_attr, jnp.zeros((pad_e, EDGE_DIM), f32)], axis=0))

    x0 = jnp.concatenate(
        [pos, prop, jnp.zeros((N, 1), f32)], axis=1)
    x0 = jnp.concatenate(
        [x0, jnp.zeros((N_PAD - N, D + 4), f32)], axis=0)
    zeros_acc = jnp.zeros((N_PAD, SW), f32)

    layers = params["layers"]

    def split_we1(p):
        we1 = p["We1"]
        return (we1[0:D], we1[D:2 * D], we1[2 * D:2 * D + 1],
                we1[2 * D + 1:])

    wd0, ws0, _, _ = split_we1(layers[0])
    a_t, b_t = _tc_prep(x0, wd0, ws0)

    xc = x0
    for l, p in enumerate(layers):
        _, _, wr, wea = split_we1(p)
        parts = []
        for h in range(H):
            gbuf = _sc_gather(a_t, b_t, dst_g[h], src_g[h])
            sbuf = _tc_edge(gbuf, ea_t, h * (E_SL // EB), wr, wea,
                            p["be1"].reshape(1, M_DIM), p["We2"],
                            p["be2"].reshape(1, M_DIM), p["Wc1"],
                            p["bc1"].reshape(1, M_DIM), p["Wc2"],
                            p["bc2"].reshape(1, 1))
            part = _sc_scatter(sbuf, dst_s[h], zeros_acc)
            parts.append(part[0])
            parts.append(part[1])
        wn1 = p["Wn1"]
        if l < N_LAYERS - 1:
            wdn, wsn, _, _ = split_we1(layers[l + 1])
            xc, a_t, b_t = _tc_node(
                xc, parts, wn1[0:D], wn1[D:],
                p["bn1"].reshape(1, M_DIM), p["Wn2"],
                p["bn2"].reshape(1, D), wdn, wsn)
        else:
            hd = params["head"]
            out = _tc_head(
                xc, parts, wn1[0:D], wn1[D:],
                p["bn1"].reshape(1, M_DIM), p["Wn2"],
                p["bn2"].reshape(1, D),
                hd["W1"], hd["b1"].reshape(1, 16),
                hd["W2"], hd["b2"].reshape(1, 2))
    return out[:N]
